# baseline (speedup 1.0000x reference)
import jax
import jax.numpy as jnp
from jax import lax
from jax.experimental import pallas as pl
from jax.experimental.pallas import tpu as pltpu
from jax.experimental.pallas import tpu_sc as plsc

N = 10000
T = 12
IN_CH = 128
HID = 64
OUT_CH = 128
E = 320000

NC = 2
NS = 16
CHUNK = 128
IBC = 32
CW = 160
EP = NS * CW * CHUNK
NTAB = 10112
ZPT = NTAB // NS


def _sc_pass_body(table, gsrc, gdst, zeros, out, tab, acc, srcbuf, dstbuf,
                  row0, row1, sem0, sem1):
  c = lax.axis_index("c")
  s = lax.axis_index("s")

  sl = pl.ds(s * ZPT, ZPT)
  pltpu.sync_copy(table.at[c, sl], tab.at[sl])
  pltpu.sync_copy(zeros.at[sl], acc.at[sl])

  plsc.subcore_barrier()

  NM = IBC

  def blk(b, carry):
    pltpu.sync_copy(gsrc.at[c, s, pl.ds(b * IBC, IBC)], srcbuf)
    pltpu.sync_copy(gdst.at[c, s, pl.ds(b * IBC, IBC)], dstbuf)
    pltpu.async_copy(tab.at[srcbuf.at[0]], row0, sem0)

    def body(i, carry2):
      j = 2 * i
      pltpu.async_copy(tab.at[srcbuf.at[j + 1]], row1, sem1)
      pltpu.make_async_copy(tab.at[srcbuf.at[j]], row0, sem0).wait()
      pltpu.sync_copy(row0, acc.at[dstbuf.at[j]], add=True)

      @pl.when(j + 2 < NM)
      def _():
        pltpu.async_copy(tab.at[srcbuf.at[j + 2]], row0, sem0)

      pltpu.make_async_copy(tab.at[srcbuf.at[j + 1]], row1, sem1).wait()
      pltpu.sync_copy(row1, acc.at[dstbuf.at[j + 1]], add=True)
      return carry2

    lax.fori_loop(0, NM // 2, body, 0)
    return carry

  lax.fori_loop(0, CW // IBC, blk, 0)
  plsc.subcore_barrier()

  pltpu.sync_copy(acc.at[sl], out.at[c, sl])


_SC_PASS_CACHE = []


def _sc_pass(table, gsrc, gdst, zeros):
  if not _SC_PASS_CACHE:
    _SC_PASS_CACHE.append(pl.kernel(
        _sc_pass_body,
        out_type=jax.ShapeDtypeStruct((NC, NTAB, HID), jnp.float32),
        mesh=plsc.VectorSubcoreMesh(
            core_axis_name="c", subcore_axis_name="s",
            num_cores=NC, num_subcores=NS),
        scratch_types=[
            pltpu.VMEM_SHARED((NTAB, HID), jnp.float32),
            pltpu.VMEM_SHARED((NTAB, HID), jnp.float32),
            pltpu.VMEM((IBC, CHUNK), jnp.int32),
            pltpu.VMEM((IBC, CHUNK), jnp.int32),
            pltpu.VMEM((CHUNK, HID), jnp.float32),
            pltpu.VMEM((CHUNK, HID), jnp.float32),
            pltpu.SemaphoreType.DMA,
            pltpu.SemaphoreType.DMA,
        ],
        compiler_params=pltpu.CompilerParams(use_tc_tiling_on_sc=False),
    ))
  return _SC_PASS_CACHE[0](table, gsrc, gdst, zeros)

def _sc_multi_body(tables, gsrc, gdst, zeros, out, tab, acc, srcbuf, dstbuf,
                   row0, row1, sem0, sem1):
  c = lax.axis_index("c")
  s = lax.axis_index("s")
  sl = pl.ds(s * ZPT, ZPT)

  def one(t, carry):
    pltpu.sync_copy(tables.at[t, c, sl], tab.at[sl])
    pltpu.sync_copy(zeros.at[sl], acc.at[sl])
    plsc.subcore_barrier()

    def blk(b, carry1):
      pltpu.sync_copy(gsrc.at[c, s, pl.ds(b * IBC, IBC)], srcbuf)
      pltpu.sync_copy(gdst.at[c, s, pl.ds(b * IBC, IBC)], dstbuf)
      pltpu.async_copy(tab.at[srcbuf.at[0]], row0, sem0)

      def body(i, carry2):
        j = 2 * i
        pltpu.async_copy(tab.at[srcbuf.at[j + 1]], row1, sem1)
        pltpu.make_async_copy(tab.at[srcbuf.at[j]], row0, sem0).wait()
        pltpu.sync_copy(row0, acc.at[dstbuf.at[j]], add=True)

        @pl.when(j + 2 < IBC)
        def _():
          pltpu.async_copy(tab.at[srcbuf.at[j + 2]], row0, sem0)

        pltpu.make_async_copy(tab.at[srcbuf.at[j + 1]], row1, sem1).wait()
        pltpu.sync_copy(row1, acc.at[dstbuf.at[j + 1]], add=True)
        return carry2

      lax.fori_loop(0, IBC // 2, body, 0)
      return carry1

    lax.fori_loop(0, CW // IBC, blk, 0)
    plsc.subcore_barrier()
    pltpu.sync_copy(acc.at[sl], out.at[t, c, sl])
    return carry

  lax.fori_loop(0, T, one, 0)


_SC_MULTI_CACHE = []


def _sc_multi(tables, gsrc, gdst, zeros):
  if not _SC_MULTI_CACHE:
    _SC_MULTI_CACHE.append(pl.kernel(
        _sc_multi_body,
        out_type=jax.ShapeDtypeStruct((T, NC, NTAB, HID), jnp.float32),
        mesh=plsc.VectorSubcoreMesh(
            core_axis_name="c", subcore_axis_name="s",
            num_cores=NC, num_subcores=NS),
        scratch_types=[
            pltpu.VMEM_SHARED((NTAB, HID), jnp.float32),
            pltpu.VMEM_SHARED((NTAB, HID), jnp.float32),
            pltpu.VMEM((IBC, CHUNK), jnp.int32),
            pltpu.VMEM((IBC, CHUNK), jnp.int32),
            pltpu.VMEM((CHUNK, HID), jnp.float32),
            pltpu.VMEM((CHUNK, HID), jnp.float32),
            pltpu.SemaphoreType.DMA,
            pltpu.SemaphoreType.DMA,
        ],
        compiler_params=pltpu.CompilerParams(use_tc_tiling_on_sc=False),
    ))
  return _SC_MULTI_CACHE[0](tables, gsrc, gdst, zeros)


BN = 1000
GRID = N // BN
BN2 = 2000
GRID2 = N // BN2


def _dis_body(z_ref, dis_ref):
  deg = z_ref[0][:, 0:1] + 1.0
  dis_ref[0] = lax.rsqrt(deg)


def _dis_kernel(zdeg):
  return pl.pallas_call(
      _dis_body,
      grid=(NC, GRID2),
      in_specs=[pl.BlockSpec((1, BN2, HID), lambda c, i: (c, i, 0))],
      out_specs=pl.BlockSpec((1, BN2, 1), lambda c, i: (c, i, 0)),
      out_shape=jax.ShapeDtypeStruct((NC, N, 1), jnp.float32),
  )(zdeg)


def _proj_body(x_ref, wp_ref, bp_ref, disf_ref, disb_ref, xp_ref, xpcat_ref):
  xq = jnp.dot(x_ref[0], wp_ref[...],
               preferred_element_type=jnp.float32) + bp_ref[...]
  xp_ref[0] = xq
  xpcat_ref[0, 0] = xq * disf_ref[...]
  xpcat_ref[0, 1] = xq * disb_ref[...]


def _proj_kernel(xs, Wp, bp, disf, disb):
  return pl.pallas_call(
      _proj_body,
      grid=(T, GRID),
      in_specs=[
          pl.BlockSpec((1, BN, IN_CH), lambda t, i: (t, i, 0)),
          pl.BlockSpec((IN_CH, HID), lambda t, i: (0, 0)),
          pl.BlockSpec((1, HID), lambda t, i: (0, 0)),
          pl.BlockSpec((BN, 1), lambda t, i: (i, 0)),
          pl.BlockSpec((BN, 1), lambda t, i: (i, 0)),
      ],
      out_specs=[
          pl.BlockSpec((1, BN, HID), lambda t, i: (t, i, 0)),
          pl.BlockSpec((1, 2, BN, HID), lambda t, i: (t, 0, i, 0)),
      ],
      out_shape=[
          jax.ShapeDtypeStruct((T, N, HID), jnp.float32),
          jax.ShapeDtypeStruct((T, 2, NTAB, HID), jnp.float32),
      ],
  )(xs, Wp, bp, disf, disb)


def _conv_gru_body(zf_ref, zb_ref, x_ref, h_ref, disf_ref, disb_ref,
                   wf_ref, bf_ref, wb_ref, bb_ref,
                   wihT_ref, bih_ref, whhT_ref, bhh_ref,
                   hout_ref, xcat_ref):
  disf = disf_ref[...]
  disb = disb_ref[...]
  x = x_ref[...]
  zf = zf_ref[0] * disf + x * (disf * disf)
  zb = zb_ref[0] * disb + x * (disb * disb)
  xc = jnp.dot(zf, wf_ref[...], preferred_element_type=jnp.float32) + bf_ref[...]
  xc = xc + jnp.dot(zb, wb_ref[...], preferred_element_type=jnp.float32) + bb_ref[...]
  xc = jnp.maximum(xc, 0.0)
  h = h_ref[...]
  gi = jnp.dot(xc, wihT_ref[...], preferred_element_type=jnp.float32) + bih_ref[...]
  gh = jnp.dot(h, whhT_ref[...], preferred_element_type=jnp.float32) + bhh_ref[...]
  r = jax.nn.sigmoid(gi[:, :HID] + gh[:, :HID])
  z = jax.nn.sigmoid(gi[:, HID:2 * HID] + gh[:, HID:2 * HID])
  n = jnp.tanh(gi[:, 2 * HID:] + r * gh[:, 2 * HID:])
  hnew = (1.0 - z) * n + z * h
  hout_ref[...] = hnew
  xcat_ref[0] = hnew * disf
  xcat_ref[1] = hnew * disb


def _conv_gru(Z, x, h, disf, disb, Wf, bf, Wb, bb, WihT, bih, WhhT, bhh):
  wspec = lambda shp: pl.BlockSpec(shp, lambda i: tuple(0 for _ in shp))
  return pl.pallas_call(
      _conv_gru_body,
      grid=(GRID,),
      in_specs=[
          pl.BlockSpec((1, BN, HID), lambda i: (0, i, 0)),
          pl.BlockSpec((1, BN, HID), lambda i: (1, i, 0)),
          pl.BlockSpec((BN, HID), lambda i: (i, 0)),
          pl.BlockSpec((BN, HID), lambda i: (i, 0)),
          pl.BlockSpec((BN, 1), lambda i: (i, 0)),
          pl.BlockSpec((BN, 1), lambda i: (i, 0)),
          wspec((HID, HID)), wspec((1, HID)),
          wspec((HID, HID)), wspec((1, HID)),
          wspec((HID, 3 * HID)), wspec((1, 3 * HID)),
          wspec((HID, 3 * HID)), wspec((1, 3 * HID)),
      ],
      out_specs=[
          pl.BlockSpec((BN, HID), lambda i: (i, 0)),
          pl.BlockSpec((2, BN, HID), lambda i: (0, i, 0)),
      ],
      out_shape=[
          jax.ShapeDtypeStruct((N, HID), jnp.float32),
          jax.ShapeDtypeStruct((2, NTAB, HID), jnp.float32),
      ],
  )(Z, Z, x, h, disf, disb, Wf, bf, Wb, bb, WihT, bih, WhhT, bhh)


def _out_body(x_ref, wo_ref, bo_ref, o_ref):
  o_ref[...] = jnp.dot(x_ref[...], wo_ref[...],
                       preferred_element_type=jnp.float32) + bo_ref[...]


def _out_kernel(xf, Wo, bo):
  return pl.pallas_call(
      _out_body,
      grid=(GRID,),
      in_specs=[
          pl.BlockSpec((BN, HID), lambda i: (i, 0)),
          pl.BlockSpec((HID, OUT_CH), lambda i: (0, 0)),
          pl.BlockSpec((1, OUT_CH), lambda i: (0, 0)),
      ],
      out_specs=pl.BlockSpec((BN, OUT_CH), lambda i: (i, 0)),
      out_shape=jax.ShapeDtypeStruct((N, OUT_CH), jnp.float32),
  )(xf, Wo, bo)


def kernel(x, edge_index, Wp, bp, Wf0, bf0, Wb0, bb0, Wih0, Whh0, bih0, bhh0,
           Wf1, bf1, Wb1, bb1, Wih1, Whh1, bih1, bhh1, Wo, bo):
  src = edge_index[0]
  dst = edge_index[1]
  of = jnp.argsort(dst)
  ob = jnp.argsort(src)
  src, dst, srcb, dstb = src[of], dst[of], src[ob], dst[ob]
  pad = EP - E
  padg = jnp.zeros((pad,), jnp.int32)
  pads = jnp.full((pad,), N, jnp.int32)
  srcp = jnp.concatenate([src, padg])
  dstp_s = jnp.concatenate([dst, pads])
  dstp_g = jnp.concatenate([dstb, padg])
  srcp_s = jnp.concatenate([srcb, pads])
  gsrc = jnp.stack([srcp, dstp_g]).reshape(NC, NS, CW, CHUNK)
  gdst = jnp.stack([dstp_s, srcp_s]).reshape(NC, NS, CW, CHUNK)

  zeros_hbm = jnp.zeros((NTAB, HID), jnp.float32)
  ones_tab = jnp.ones((NC, NTAB, HID), jnp.float32)

  zdeg = _sc_pass(ones_tab, gsrc, gdst, zeros_hbm)
  discat = _dis_kernel(zdeg[:, :N])
  disf = discat[0]
  disb = discat[1]

  xs = x.reshape(T, N, IN_CH)
  xp, xpcat = _proj_kernel(xs, Wp, bp.reshape(1, HID), disf, disb)

  layer_params = [
      (Wf0, bf0.reshape(1, HID), Wb0, bb0.reshape(1, HID),
       Wih0.T, bih0.reshape(1, 3 * HID), Whh0.T, bhh0.reshape(1, 3 * HID)),
      (Wf1, bf1.reshape(1, HID), Wb1, bb1.reshape(1, HID),
       Wih1.T, bih1.reshape(1, 3 * HID), Whh1.T, bhh1.reshape(1, 3 * HID)),
  ]

  h = [jnp.zeros((N, HID), jnp.float32), jnp.zeros((N, HID), jnp.float32)]
  z0s = [_sc_pass(xpcat[t], gsrc, gdst, zeros_hbm) for t in range(T)]
  cur = None
  for t in range(T):
    cur = xp[t]
    Z = z0s[t]
    for l in range(2):
      Wf, bf, Wb, bb, WihT, bih, WhhT, bhh = layer_params[l]
      hnew, xcat = _conv_gru(Z, cur, h[l], disf, disb,
                             Wf, bf, Wb, bb, WihT, bih, WhhT, bhh)
      h[l] = hnew
      cur = hnew
      if l == 0:
        Z = _sc_pass(xcat, gsrc, gdst, zeros_hbm)
  out = _out_kernel(cur, Wo, bo.reshape(1, OUT_CH))
  return out.reshape(1, N, OUT_CH)


# speedup vs baseline: 1.1939x; 1.1939x over previous
import jax
import jax.numpy as jnp
from jax import lax
from jax.experimental import pallas as pl
from jax.experimental.pallas import tpu as pltpu
from jax.experimental.pallas import tpu_sc as plsc

N = 10000
T = 12
IN_CH = 128
HID = 64
OUT_CH = 128
E = 320000

NC = 2
NS = 16
CHUNK = 128
IBC = 32
CW = 160
EP = NS * CW * CHUNK
NTAB = 10112
ZPT = NTAB // NS


def _sc_pass_body(table, gsrc, gdst, zeros, out, tab, acc, srcbuf, dstbuf,
                  row0, row1, sem0, sem1):
  c = lax.axis_index("c")
  s = lax.axis_index("s")

  sl = pl.ds(s * ZPT, ZPT)
  pltpu.sync_copy(table.at[c, sl], tab.at[sl])
  pltpu.sync_copy(zeros.at[sl], acc.at[sl])

  plsc.subcore_barrier()

  NM = IBC

  def blk(b, carry):
    pltpu.sync_copy(gsrc.at[c, s, pl.ds(b * IBC, IBC)], srcbuf)
    pltpu.sync_copy(gdst.at[c, s, pl.ds(b * IBC, IBC)], dstbuf)
    pltpu.async_copy(tab.at[srcbuf.at[0]], row0, sem0)

    def body(i, carry2):
      j = 2 * i
      pltpu.async_copy(tab.at[srcbuf.at[j + 1]], row1, sem1)
      pltpu.make_async_copy(tab.at[srcbuf.at[j]], row0, sem0).wait()
      pltpu.sync_copy(row0, acc.at[dstbuf.at[j]], add=True)

      @pl.when(j + 2 < NM)
      def _():
        pltpu.async_copy(tab.at[srcbuf.at[j + 2]], row0, sem0)

      pltpu.make_async_copy(tab.at[srcbuf.at[j + 1]], row1, sem1).wait()
      pltpu.sync_copy(row1, acc.at[dstbuf.at[j + 1]], add=True)
      return carry2

    lax.fori_loop(0, NM // 2, body, 0)
    return carry

  lax.fori_loop(0, CW // IBC, blk, 0)
  plsc.subcore_barrier()

  pltpu.sync_copy(acc.at[sl], out.at[c, sl])


_SC_PASS_CACHE = []


def _sc_pass(table, gsrc, gdst, zeros):
  if not _SC_PASS_CACHE:
    _SC_PASS_CACHE.append(pl.kernel(
        _sc_pass_body,
        out_type=jax.ShapeDtypeStruct((NC, NTAB, HID), jnp.float32),
        mesh=plsc.VectorSubcoreMesh(
            core_axis_name="c", subcore_axis_name="s",
            num_cores=NC, num_subcores=NS),
        scratch_types=[
            pltpu.VMEM_SHARED((NTAB, HID), jnp.float32),
            pltpu.VMEM_SHARED((NTAB, HID), jnp.float32),
            pltpu.VMEM((IBC, CHUNK), jnp.int32),
            pltpu.VMEM((IBC, CHUNK), jnp.int32),
            pltpu.VMEM((CHUNK, HID), jnp.float32),
            pltpu.VMEM((CHUNK, HID), jnp.float32),
            pltpu.SemaphoreType.DMA,
            pltpu.SemaphoreType.DMA,
        ],
        compiler_params=pltpu.CompilerParams(use_tc_tiling_on_sc=False),
    ))
  return _SC_PASS_CACHE[0](table, gsrc, gdst, zeros)

def _sc_multi_body(tables, gsrc, gdst, zeros, out, tab, acc, srcbuf, dstbuf,
                   row0, row1, sem0, sem1):
  c = lax.axis_index("c")
  s = lax.axis_index("s")
  sl = pl.ds(s * ZPT, ZPT)

  def one(t, carry):
    pltpu.sync_copy(tables.at[t, c, sl], tab.at[sl])
    pltpu.sync_copy(zeros.at[sl], acc.at[sl])
    plsc.subcore_barrier()

    def blk(b, carry1):
      pltpu.sync_copy(gsrc.at[c, s, pl.ds(b * IBC, IBC)], srcbuf)
      pltpu.sync_copy(gdst.at[c, s, pl.ds(b * IBC, IBC)], dstbuf)
      pltpu.async_copy(tab.at[srcbuf.at[0]], row0, sem0)

      def body(i, carry2):
        j = 2 * i
        pltpu.async_copy(tab.at[srcbuf.at[j + 1]], row1, sem1)
        pltpu.make_async_copy(tab.at[srcbuf.at[j]], row0, sem0).wait()
        pltpu.sync_copy(row0, acc.at[dstbuf.at[j]], add=True)

        @pl.when(j + 2 < IBC)
        def _():
          pltpu.async_copy(tab.at[srcbuf.at[j + 2]], row0, sem0)

        pltpu.make_async_copy(tab.at[srcbuf.at[j + 1]], row1, sem1).wait()
        pltpu.sync_copy(row1, acc.at[dstbuf.at[j + 1]], add=True)
        return carry2

      lax.fori_loop(0, IBC // 2, body, 0)
      return carry1

    lax.fori_loop(0, CW // IBC, blk, 0)
    plsc.subcore_barrier()
    pltpu.sync_copy(acc.at[sl], out.at[t, c, sl])
    return carry

  lax.fori_loop(0, T, one, 0)


_SC_MULTI_CACHE = []


def _sc_multi(tables, gsrc, gdst, zeros):
  if not _SC_MULTI_CACHE:
    _SC_MULTI_CACHE.append(pl.kernel(
        _sc_multi_body,
        out_type=jax.ShapeDtypeStruct((T, NC, NTAB, HID), jnp.float32),
        mesh=plsc.VectorSubcoreMesh(
            core_axis_name="c", subcore_axis_name="s",
            num_cores=NC, num_subcores=NS),
        scratch_types=[
            pltpu.VMEM_SHARED((NTAB, HID), jnp.float32),
            pltpu.VMEM_SHARED((NTAB, HID), jnp.float32),
            pltpu.VMEM((IBC, CHUNK), jnp.int32),
            pltpu.VMEM((IBC, CHUNK), jnp.int32),
            pltpu.VMEM((CHUNK, HID), jnp.float32),
            pltpu.VMEM((CHUNK, HID), jnp.float32),
            pltpu.SemaphoreType.DMA,
            pltpu.SemaphoreType.DMA,
        ],
        compiler_params=pltpu.CompilerParams(use_tc_tiling_on_sc=False),
    ))
  return _SC_MULTI_CACHE[0](tables, gsrc, gdst, zeros)


BN = 1000
GRID = N // BN
BN2 = 2000
GRID2 = N // BN2


def _dis_body(z_ref, dis_ref):
  deg = z_ref[0][:, 0:1] + 1.0
  dis_ref[0] = lax.rsqrt(deg)


def _dis_kernel(zdeg):
  return pl.pallas_call(
      _dis_body,
      grid=(NC, GRID2),
      in_specs=[pl.BlockSpec((1, BN2, HID), lambda c, i: (c, i, 0))],
      out_specs=pl.BlockSpec((1, BN2, 1), lambda c, i: (c, i, 0)),
      out_shape=jax.ShapeDtypeStruct((NC, N, 1), jnp.float32),
  )(zdeg)


def _proj_body(x_ref, wp_ref, bp_ref, disf_ref, disb_ref, xp_ref, xpcat_ref):
  xq = jnp.dot(x_ref[0], wp_ref[...],
               preferred_element_type=jnp.float32) + bp_ref[...]
  xp_ref[0] = xq
  xpcat_ref[0, 0] = xq * disf_ref[...]
  xpcat_ref[0, 1] = xq * disb_ref[...]


def _proj_kernel(xs, Wp, bp, disf, disb):
  return pl.pallas_call(
      _proj_body,
      grid=(T, GRID),
      in_specs=[
          pl.BlockSpec((1, BN, IN_CH), lambda t, i: (t, i, 0)),
          pl.BlockSpec((IN_CH, HID), lambda t, i: (0, 0)),
          pl.BlockSpec((1, HID), lambda t, i: (0, 0)),
          pl.BlockSpec((BN, 1), lambda t, i: (i, 0)),
          pl.BlockSpec((BN, 1), lambda t, i: (i, 0)),
      ],
      out_specs=[
          pl.BlockSpec((1, BN, HID), lambda t, i: (t, i, 0)),
          pl.BlockSpec((1, 2, BN, HID), lambda t, i: (t, 0, i, 0)),
      ],
      out_shape=[
          jax.ShapeDtypeStruct((T, N, HID), jnp.float32),
          jax.ShapeDtypeStruct((T, 2, NTAB, HID), jnp.float32),
      ],
  )(xs, Wp, bp, disf, disb)


def _conv_gru_body(zf_ref, zb_ref, x_ref, h_ref, disf_ref, disb_ref,
                   wf_ref, bf_ref, wb_ref, bb_ref,
                   wihT_ref, bih_ref, whhT_ref, bhh_ref,
                   hout_ref, xcat_ref):
  disf = disf_ref[...]
  disb = disb_ref[...]
  x = x_ref[...]
  zf = zf_ref[0] * disf + x * (disf * disf)
  zb = zb_ref[0] * disb + x * (disb * disb)
  xc = jnp.dot(zf, wf_ref[...], preferred_element_type=jnp.float32) + bf_ref[...]
  xc = xc + jnp.dot(zb, wb_ref[...], preferred_element_type=jnp.float32) + bb_ref[...]
  xc = jnp.maximum(xc, 0.0)
  h = h_ref[...]
  gi = jnp.dot(xc, wihT_ref[...], preferred_element_type=jnp.float32) + bih_ref[...]
  gh = jnp.dot(h, whhT_ref[...], preferred_element_type=jnp.float32) + bhh_ref[...]
  r = jax.nn.sigmoid(gi[:, :HID] + gh[:, :HID])
  z = jax.nn.sigmoid(gi[:, HID:2 * HID] + gh[:, HID:2 * HID])
  n = jnp.tanh(gi[:, 2 * HID:] + r * gh[:, 2 * HID:])
  hnew = (1.0 - z) * n + z * h
  hout_ref[...] = hnew
  xcat_ref[0] = hnew * disf
  xcat_ref[1] = hnew * disb


def _conv_gru(Z, x, h, disf, disb, Wf, bf, Wb, bb, WihT, bih, WhhT, bhh):
  wspec = lambda shp: pl.BlockSpec(shp, lambda i: tuple(0 for _ in shp))
  return pl.pallas_call(
      _conv_gru_body,
      grid=(GRID,),
      in_specs=[
          pl.BlockSpec((1, BN, HID), lambda i: (0, i, 0)),
          pl.BlockSpec((1, BN, HID), lambda i: (1, i, 0)),
          pl.BlockSpec((BN, HID), lambda i: (i, 0)),
          pl.BlockSpec((BN, HID), lambda i: (i, 0)),
          pl.BlockSpec((BN, 1), lambda i: (i, 0)),
          pl.BlockSpec((BN, 1), lambda i: (i, 0)),
          wspec((HID, HID)), wspec((1, HID)),
          wspec((HID, HID)), wspec((1, HID)),
          wspec((HID, 3 * HID)), wspec((1, 3 * HID)),
          wspec((HID, 3 * HID)), wspec((1, 3 * HID)),
      ],
      out_specs=[
          pl.BlockSpec((BN, HID), lambda i: (i, 0)),
          pl.BlockSpec((2, BN, HID), lambda i: (0, i, 0)),
      ],
      out_shape=[
          jax.ShapeDtypeStruct((N, HID), jnp.float32),
          jax.ShapeDtypeStruct((2, NTAB, HID), jnp.float32),
      ],
  )(Z, Z, x, h, disf, disb, Wf, bf, Wb, bb, WihT, bih, WhhT, bhh)


def _out_body(x_ref, wo_ref, bo_ref, o_ref):
  o_ref[...] = jnp.dot(x_ref[...], wo_ref[...],
                       preferred_element_type=jnp.float32) + bo_ref[...]


def _out_kernel(xf, Wo, bo):
  return pl.pallas_call(
      _out_body,
      grid=(GRID,),
      in_specs=[
          pl.BlockSpec((BN, HID), lambda i: (i, 0)),
          pl.BlockSpec((HID, OUT_CH), lambda i: (0, 0)),
          pl.BlockSpec((1, OUT_CH), lambda i: (0, 0)),
      ],
      out_specs=pl.BlockSpec((BN, OUT_CH), lambda i: (i, 0)),
      out_shape=jax.ShapeDtypeStruct((N, OUT_CH), jnp.float32),
  )(xf, Wo, bo)


def kernel(x, edge_index, Wp, bp, Wf0, bf0, Wb0, bb0, Wih0, Whh0, bih0, bhh0,
           Wf1, bf1, Wb1, bb1, Wih1, Whh1, bih1, bhh1, Wo, bo):
  src = edge_index[0]
  dst = edge_index[1]
  pad = EP - E
  padg = jnp.zeros((pad,), jnp.int32)
  pads = jnp.full((pad,), N, jnp.int32)
  srcp = jnp.concatenate([src, padg])
  dstp_s = jnp.concatenate([dst, pads])
  dstp_g = jnp.concatenate([dst, padg])
  srcp_s = jnp.concatenate([src, pads])
  gsrc = jnp.stack([srcp, dstp_g]).reshape(NC, NS, CW, CHUNK)
  gdst = jnp.stack([dstp_s, srcp_s]).reshape(NC, NS, CW, CHUNK)

  zeros_hbm = jnp.zeros((NTAB, HID), jnp.float32)
  ones_tab = jnp.ones((NC, NTAB, HID), jnp.float32)

  zdeg = _sc_pass(ones_tab, gsrc, gdst, zeros_hbm)
  discat = _dis_kernel(zdeg[:, :N])
  disf = discat[0]
  disb = discat[1]

  xs = x.reshape(T, N, IN_CH)
  xp, xpcat = _proj_kernel(xs, Wp, bp.reshape(1, HID), disf, disb)

  layer_params = [
      (Wf0, bf0.reshape(1, HID), Wb0, bb0.reshape(1, HID),
       Wih0.T, bih0.reshape(1, 3 * HID), Whh0.T, bhh0.reshape(1, 3 * HID)),
      (Wf1, bf1.reshape(1, HID), Wb1, bb1.reshape(1, HID),
       Wih1.T, bih1.reshape(1, 3 * HID), Whh1.T, bhh1.reshape(1, 3 * HID)),
  ]

  h = [jnp.zeros((N, HID), jnp.float32), jnp.zeros((N, HID), jnp.float32)]
  z0 = [None] * T
  z0[0] = _sc_pass(xpcat[0], gsrc, gdst, zeros_hbm)
  z0[1] = _sc_pass(xpcat[1], gsrc, gdst, zeros_hbm)
  cur = None
  for t in range(T):
    cur = xp[t]
    Z = z0[t]
    for l in range(2):
      Wf, bf, Wb, bb, WihT, bih, WhhT, bhh = layer_params[l]
      hnew, xcat = _conv_gru(Z, cur, h[l], disf, disb,
                             Wf, bf, Wb, bb, WihT, bih, WhhT, bhh)
      h[l] = hnew
      cur = hnew
      if l == 0:
        if t + 2 < T:
          z0[t + 2] = _sc_pass(xpcat[t + 2], gsrc, gdst, zeros_hbm)
        Z = _sc_pass(xcat, gsrc, gdst, zeros_hbm)
  out = _out_kernel(cur, Wo, bo.reshape(1, OUT_CH))
  return out.reshape(1, N, OUT_CH)
